# baseline (speedup 1.0000x reference)
import functools

import jax
import jax.numpy as jnp
from jax import lax
from jax.experimental import pallas as pl
from jax.experimental.pallas import tpu as pltpu
from jax.experimental.pallas import tpu_sc as plsc

USERS = 1000000
ITEMS = 1000000
BATCH = 16384
EMBED = 16
GPR = 8
GCOLS = EMBED * GPR
NC = 2
NS = 16
L = 16
NW = NC * NS
BPW = BATCH // NW
CHUNK = 128
HALF = BPW // 2


def _mf_body(user_hbm, item_hbm, u_hbm, i_hbm, out_hbm,
             uidx_v, iidx_v, ugid_v, igid_v, ugrp_v, igrp_v, out_v, sem):
    wid = lax.axis_index("s") * NC + lax.axis_index("c")
    base = wid * BPW
    pltpu.sync_copy(user_hbm.at[pl.ds(base, BPW)], uidx_v)
    pltpu.sync_copy(item_hbm.at[pl.ds(base, BPW)], iidx_v)

    def gids(k, carry):
        sl = pl.ds(k * L, L)
        ugid_v[sl] = lax.shift_right_logical(uidx_v[sl], 3)
        igid_v[sl] = lax.shift_right_logical(iidx_v[sl], 3)
        return carry

    lax.fori_loop(0, BPW // L, gids, 0)

    iot = lax.iota(jnp.int32, L)

    for h in range(BPW // HALF):
        copies = []
        for j in range(HALF // CHUNK):
            src = pl.ds(h * HALF + j * CHUNK, CHUNK)
            dst = pl.ds(j * CHUNK, CHUNK)
            copies.append(
                pltpu.async_copy(u_hbm.at[ugid_v.at[src]], ugrp_v.at[dst], sem))
            copies.append(
                pltpu.async_copy(i_hbm.at[igid_v.at[src]], igrp_v.at[dst], sem))
        for c in copies:
            c.wait()

        def group(g, carry):
            b0 = h * HALF + g * L
            rows = g * L + iot
            ucol0 = (uidx_v[pl.ds(b0, L)] & 7) * EMBED
            icol0 = (iidx_v[pl.ds(b0, L)] & 7) * EMBED
            acc = jnp.zeros((L,), jnp.float32)
            for d in range(EMBED):
                uu = plsc.load_gather(ugrp_v, [rows, ucol0 + d])
                ii = plsc.load_gather(igrp_v, [rows, icol0 + d])
                acc = acc + uu * ii
            out_v[pl.ds(b0, L)] = acc
            return carry

        lax.fori_loop(0, HALF // L, group, 0, unroll=2)

    pltpu.sync_copy(out_v, out_hbm.at[pl.ds(base, BPW)])


def kernel(user, item, U, I):
    user = user.astype(jnp.int32)
    item = item.astype(jnp.int32)
    Ug = U.reshape(USERS // GPR, GCOLS)
    Ig = I.reshape(ITEMS // GPR, GCOLS)
    mesh = plsc.VectorSubcoreMesh(core_axis_name="c", subcore_axis_name="s")
    k = functools.partial(
        pl.kernel,
        out_type=jax.ShapeDtypeStruct((BATCH,), jnp.float32),
        mesh=mesh,
        compiler_params=pltpu.CompilerParams(
            needs_layout_passes=False, use_tc_tiling_on_sc=True
        ),
        scratch_types=[
            pltpu.VMEM((BPW,), jnp.int32),
            pltpu.VMEM((BPW,), jnp.int32),
            pltpu.VMEM((BPW,), jnp.int32),
            pltpu.VMEM((BPW,), jnp.int32),
            pltpu.VMEM((HALF, GCOLS), jnp.float32),
            pltpu.VMEM((HALF, GCOLS), jnp.float32),
            pltpu.VMEM((BPW,), jnp.float32),
            pltpu.SemaphoreType.DMA,
        ],
    )(_mf_body)
    return k(user, item, Ug, Ig)


# speedup vs baseline: 5.7504x; 5.7504x over previous
import functools

import jax
import jax.numpy as jnp
from jax import lax
from jax.experimental import pallas as pl
from jax.experimental.pallas import tpu as pltpu
from jax.experimental.pallas import tpu_sc as plsc

BATCH = 16384
EMBED = 16
NC = 2
NS = 16
L = 16
NW = NC * NS
BPW = BATCH // NW
CQ = 16
NCHUNK = BPW // CQ
W = 128


def _win_starts(idx):
    return idx - (idx & 127)


def _win_offs(idx):
    return idx & 127


def _mf_body(user_hbm, item_hbm, ut_hbm, it_hbm, out_hbm,
             uidx_v, iidx_v, ucols_v, icols_v, out_v, sem):
    wid = lax.axis_index("s") * NC + lax.axis_index("c")
    base = wid * BPW
    pltpu.sync_copy(user_hbm.at[pl.ds(base, BPW)], uidx_v)
    pltpu.sync_copy(item_hbm.at[pl.ds(base, BPW)], iidx_v)

    iot = lax.iota(jnp.int32, L)

    for chunk in range(NCHUNK):
        def fire(g, carry):
            qb = chunk * CQ + g * L
            ustart = _win_starts(uidx_v[pl.ds(qb, L)])
            istart = _win_starts(iidx_v[pl.ds(qb, L)])
            for j in range(L):
                dst = pl.ds((g * L + j) * W, W)
                pltpu.async_copy(
                    ut_hbm.at[:, pl.ds(pl.multiple_of(ustart[j], 128), W)],
                    ucols_v.at[:, dst], sem)
                pltpu.async_copy(
                    it_hbm.at[:, pl.ds(pl.multiple_of(istart[j], 128), W)],
                    icols_v.at[:, dst], sem)
            return carry

        lax.fori_loop(0, CQ // L, fire, 0)

        pltpu.make_async_copy(ut_hbm.at[:, pl.ds(0, CQ * W)], ucols_v, sem).wait()
        pltpu.make_async_copy(it_hbm.at[:, pl.ds(0, CQ * W)], icols_v, sem).wait()

        def comp(g, carry):
            qb = chunk * CQ + g * L
            ucol = g * L * W + iot * W + _win_offs(uidx_v[pl.ds(qb, L)])
            icol = g * L * W + iot * W + _win_offs(iidx_v[pl.ds(qb, L)])
            acc = jnp.zeros((L,), jnp.float32)
            for d in range(EMBED):
                drow = jnp.full((L,), d, jnp.int32)
                uu = plsc.load_gather(ucols_v, [drow, ucol])
                ii = plsc.load_gather(icols_v, [drow, icol])
                acc = acc + uu * ii
            out_v[pl.ds(qb, L)] = acc
            return carry

        lax.fori_loop(0, CQ // L, comp, 0)

    pltpu.sync_copy(out_v, out_hbm.at[pl.ds(base, BPW)])


def kernel(user, item, U, I):
    user = user.astype(jnp.int32)
    item = item.astype(jnp.int32)
    Ut = U.T
    It = I.T
    mesh = plsc.VectorSubcoreMesh(core_axis_name="c", subcore_axis_name="s")
    k = functools.partial(
        pl.kernel,
        out_type=jax.ShapeDtypeStruct((BATCH,), jnp.float32),
        mesh=mesh,
        compiler_params=pltpu.CompilerParams(
            needs_layout_passes=False, use_tc_tiling_on_sc=True
        ),
        scratch_types=[
            pltpu.VMEM((BPW,), jnp.int32),
            pltpu.VMEM((BPW,), jnp.int32),
            pltpu.VMEM((EMBED, CQ * W), jnp.float32),
            pltpu.VMEM((EMBED, CQ * W), jnp.float32),
            pltpu.VMEM((BPW,), jnp.float32),
            pltpu.SemaphoreType.DMA,
        ],
    )(_mf_body)
    return k(user, item, Ut, It)


# speedup vs baseline: 6.1693x; 1.0728x over previous
import functools

import jax
import jax.numpy as jnp
from jax import lax
from jax.experimental import pallas as pl
from jax.experimental.pallas import tpu as pltpu
from jax.experimental.pallas import tpu_sc as plsc

BATCH = 16384
EMBED = 16
NC = 2
NS = 16
L = 16
NW = NC * NS
BPW = BATCH // NW
CQ = 8
NCHUNK = BPW // CQ
W = 128
CW = CQ * W


def _mf_body(user_hbm, item_hbm, ut_hbm, it_hbm, out_hbm,
             uidx_v, iidx_v, ucols_v, icols_v, out_v, sems):
    wid = lax.axis_index("s") * NC + lax.axis_index("c")
    base = wid * BPW
    pltpu.sync_copy(user_hbm.at[pl.ds(base, BPW)], uidx_v.at[pl.ds(0, BPW)])
    pltpu.sync_copy(item_hbm.at[pl.ds(base, BPW)], iidx_v.at[pl.ds(0, BPW)])

    iot = lax.iota(jnp.int32, L)

    def fire(k, p):
        uvec = uidx_v[pl.ds(k * CQ, L)]
        ivec = iidx_v[pl.ds(k * CQ, L)]
        ustart = uvec & ~127
        istart = ivec & ~127
        sem = sems.at[p]
        for j in range(CQ):
            dst = pl.ds(p * CW + j * W, W)
            pltpu.async_copy(
                ut_hbm.at[:, pl.ds(pl.multiple_of(ustart[j], 128), W)],
                ucols_v.at[:, dst], sem)
            pltpu.async_copy(
                it_hbm.at[:, pl.ds(pl.multiple_of(istart[j], 128), W)],
                icols_v.at[:, dst], sem)

    def drain(p):
        sem = sems.at[p]
        pltpu.make_async_copy(
            ut_hbm.at[:, pl.ds(0, CW)], ucols_v.at[:, pl.ds(0, CW)], sem).wait()
        pltpu.make_async_copy(
            it_hbm.at[:, pl.ds(0, CW)], icols_v.at[:, pl.ds(0, CW)], sem).wait()

    def comp(k, p):
        uvec = uidx_v[pl.ds(k * CQ, L)]
        ivec = iidx_v[pl.ds(k * CQ, L)]
        cbase = p * CW + (iot & (CQ - 1)) * W
        ucol = cbase + (uvec & 127)
        icol = cbase + (ivec & 127)
        acc = jnp.zeros((L,), jnp.float32)
        for d in range(EMBED):
            drow = jnp.full((L,), d, jnp.int32)
            uu = plsc.load_gather(ucols_v, [drow, ucol])
            ii = plsc.load_gather(icols_v, [drow, icol])
            acc = acc + uu * ii
        out_v[pl.ds(k * CQ, L)] = acc

    fire(0, 0)
    fire(1, 1)

    def step(g, carry):
        p = g & 1
        drain(p)
        comp(g, p)
        fire(g + 2, p)
        return carry

    lax.fori_loop(0, NCHUNK - 2, step, 0)
    drain(0)
    comp(NCHUNK - 2, 0)
    drain(1)
    comp(NCHUNK - 1, 1)

    pltpu.sync_copy(out_v.at[pl.ds(0, BPW)], out_hbm.at[pl.ds(base, BPW)])


def kernel(user, item, U, I):
    user = user.astype(jnp.int32)
    item = item.astype(jnp.int32)
    Ut = U.T
    It = I.T
    mesh = plsc.VectorSubcoreMesh(core_axis_name="c", subcore_axis_name="s")
    k = functools.partial(
        pl.kernel,
        out_type=jax.ShapeDtypeStruct((BATCH,), jnp.float32),
        mesh=mesh,
        compiler_params=pltpu.CompilerParams(
            needs_layout_passes=False, use_tc_tiling_on_sc=True
        ),
        scratch_types=[
            pltpu.VMEM((BPW + L,), jnp.int32),
            pltpu.VMEM((BPW + L,), jnp.int32),
            pltpu.VMEM((EMBED, 2 * CW), jnp.float32),
            pltpu.VMEM((EMBED, 2 * CW), jnp.float32),
            pltpu.VMEM((BPW + L,), jnp.float32),
            pltpu.SemaphoreType.DMA((2,)),
        ],
    )(_mf_body)
    return k(user, item, Ut, It)


# speedup vs baseline: 6.7056x; 1.0869x over previous
import functools

import jax
import jax.numpy as jnp
from jax import lax
from jax.experimental import pallas as pl
from jax.experimental.pallas import tpu as pltpu
from jax.experimental.pallas import tpu_sc as plsc

BATCH = 16384
EMBED = 16
NC = 2
NS = 16
L = 16
NW = NC * NS
BPW = BATCH // NW
CQ = 8
NCHUNK = BPW // CQ
W = 128
CW = CQ * W
DEPTH = 3


def _mf_body(user_hbm, item_hbm, ut_hbm, it_hbm, out_hbm,
             uidx_v, iidx_v, ucols_v, icols_v, out_v, sems):
    wid = lax.axis_index("s") * NC + lax.axis_index("c")
    base = wid * BPW
    pltpu.sync_copy(user_hbm.at[pl.ds(base, BPW)], uidx_v.at[pl.ds(0, BPW)])
    pltpu.sync_copy(item_hbm.at[pl.ds(base, BPW)], iidx_v.at[pl.ds(0, BPW)])

    iot = lax.iota(jnp.int32, L)

    def fire(k, p):
        uvec = uidx_v[pl.ds(k * CQ, L)]
        ivec = iidx_v[pl.ds(k * CQ, L)]
        ustart = uvec & ~127
        istart = ivec & ~127
        sem = sems.at[p]
        for j in range(CQ):
            dst = pl.ds(p * CW + j * W, W)
            pltpu.async_copy(
                ut_hbm.at[:, pl.ds(pl.multiple_of(ustart[j], 128), W)],
                ucols_v.at[:, dst], sem)
            pltpu.async_copy(
                it_hbm.at[:, pl.ds(pl.multiple_of(istart[j], 128), W)],
                icols_v.at[:, dst], sem)

    def drain(p):
        sem = sems.at[p]
        pltpu.make_async_copy(
            ut_hbm.at[:, pl.ds(0, CW)], ucols_v.at[:, pl.ds(0, CW)], sem).wait()
        pltpu.make_async_copy(
            it_hbm.at[:, pl.ds(0, CW)], icols_v.at[:, pl.ds(0, CW)], sem).wait()

    def comp(k, p):
        uvec = uidx_v[pl.ds(k * CQ, L)]
        ivec = iidx_v[pl.ds(k * CQ, L)]
        cbase = p * CW + (iot & (CQ - 1)) * W
        ucol = cbase + (uvec & 127)
        icol = cbase + (ivec & 127)
        acc = jnp.zeros((L,), jnp.float32)
        for d in range(EMBED):
            drow = jnp.full((L,), d, jnp.int32)
            uu = plsc.load_gather(ucols_v, [drow, ucol])
            ii = plsc.load_gather(icols_v, [drow, icol])
            acc = acc + uu * ii
        out_v[pl.ds(k * CQ, L)] = acc

    for p in range(DEPTH):
        fire(p, p)

    def step(g, carry):
        p = lax.rem(g, DEPTH)
        drain(p)
        comp(g, p)
        fire(g + DEPTH, p)
        return carry

    lax.fori_loop(0, NCHUNK - DEPTH, step, 0)
    for k in range(NCHUNK - DEPTH, NCHUNK):
        p = k % DEPTH
        drain(p)
        comp(k, p)

    pltpu.sync_copy(out_v.at[pl.ds(0, BPW)], out_hbm.at[pl.ds(base, BPW)])


def kernel(user, item, U, I):
    user = user.astype(jnp.int32)
    item = item.astype(jnp.int32)
    Ut = U.T
    It = I.T
    mesh = plsc.VectorSubcoreMesh(core_axis_name="c", subcore_axis_name="s")
    k = functools.partial(
        pl.kernel,
        out_type=jax.ShapeDtypeStruct((BATCH,), jnp.float32),
        mesh=mesh,
        compiler_params=pltpu.CompilerParams(
            needs_layout_passes=False, use_tc_tiling_on_sc=True
        ),
        scratch_types=[
            pltpu.VMEM((BPW + L,), jnp.int32),
            pltpu.VMEM((BPW + L,), jnp.int32),
            pltpu.VMEM((EMBED, DEPTH * CW), jnp.float32),
            pltpu.VMEM((EMBED, DEPTH * CW), jnp.float32),
            pltpu.VMEM((BPW + L,), jnp.float32),
            pltpu.SemaphoreType.DMA((DEPTH,)),
        ],
    )(_mf_body)
    return k(user, item, Ut, It)
